# baseline (speedup 1.0000x reference)
import functools

import jax
import jax.numpy as jnp
from jax import lax
from jax.experimental import pallas as pl
from jax.experimental.pallas import tpu as pltpu
from jax.experimental.pallas import tpu_sc as plsc

NUM_CLASSES = 1000000
DROPOUT_PROB = 0.1

_info = plsc.get_sparse_core_info()
_NC, _NS = _info.num_cores, _info.num_subcores
_NW = _NC * _NS
_L = 16
_OSTRIDE = 128


def _gather_kernel(idx_hbm, table_t_hbm, out_hbm, idx_v, blk_v, slots, osem):
    wid = lax.axis_index("s") * _NC + lax.axis_index("c")
    b_per_w = idx_v.shape[0]
    D = table_t_hbm.shape[0]
    base = wid * b_per_w
    pltpu.sync_copy(idx_hbm.at[pl.ds(base, b_per_w)], idx_v)

    def per_group(g, carry):
        vec = idx_v[pl.ds(g * _L, _L)]
        for j in range(_L):
            i = g * _L + j
            r = vec[j]
            blk = pl.multiple_of((r >> 7) * 128, 128)
            l = r & 127
            pltpu.sync_copy(table_t_hbm.at[:, pl.ds(blk, 128)], blk_v)
            sbase = (i & 3) * _OSTRIDE

            @pl.when(i >= 4)
            def _():
                pltpu.make_async_copy(
                    slots.at[pl.ds(0, _OSTRIDE)],
                    out_hbm.at[pl.ds(0, _OSTRIDE)],
                    osem,
                ).wait()

            for q in range(D // _L):
                v = plsc.load_gather(
                    blk_v,
                    [jnp.arange(_L, dtype=jnp.int32) + q * _L,
                     jnp.full((_L,), l, jnp.int32)],
                )
                slots[pl.ds(sbase + q * _L, _L)] = v
            pltpu.async_copy(
                slots.at[pl.ds(sbase, _OSTRIDE)],
                out_hbm.at[pl.ds((base + i) * _OSTRIDE, _OSTRIDE)],
                osem,
            )
        return carry

    lax.fori_loop(0, b_per_w // _L, per_group, jnp.int32(0))
    for k in range(4):
        pltpu.make_async_copy(
            slots.at[pl.ds(0, _OSTRIDE)],
            out_hbm.at[pl.ds(0, _OSTRIDE)],
            osem,
        ).wait()


def kernel(labels, train, embedding_table):
    B = labels.shape[0]
    D = embedding_table.shape[1]
    key = jax.random.key(42)
    drop_ids = jax.random.uniform(key, (B,)) < DROPOUT_PROB
    dropped = jnp.where(drop_ids, NUM_CLASSES, labels)
    idx = jnp.where(train != 0, dropped, labels).astype(jnp.int32)

    b_per_w = B // _NW
    table_t = embedding_table.T

    mesh = plsc.VectorSubcoreMesh(core_axis_name="c", subcore_axis_name="s")
    run = functools.partial(
        pl.kernel,
        _gather_kernel,
        mesh=mesh,
        compiler_params=pltpu.CompilerParams(needs_layout_passes=False),
        out_type=jax.ShapeDtypeStruct((B * _OSTRIDE,), jnp.float32),
        scratch_types=[
            pltpu.VMEM((b_per_w,), jnp.int32),
            pltpu.VMEM((D, 128), jnp.float32),
            pltpu.VMEM((4 * _OSTRIDE,), jnp.float32),
            pltpu.SemaphoreType.DMA,
        ],
    )()
    out_flat = run(idx, table_t)
    return out_flat.reshape(B, _OSTRIDE)[:, :D]


# speedup vs baseline: 1.2767x; 1.2767x over previous
import functools

import jax
import jax.numpy as jnp
from jax import lax
from jax.experimental import pallas as pl
from jax.experimental.pallas import tpu as pltpu
from jax.experimental.pallas import tpu_sc as plsc

NUM_CLASSES = 1000000
DROPOUT_PROB = 0.1

_info = plsc.get_sparse_core_info()
_NC, _NS = _info.num_cores, _info.num_subcores
_NW = _NC * _NS

_L = 16
_CL = 1024
_OSTRIDE = 128


def _gather_kernel(idx_hbm, table_t_hbm, out_hbm, all_idx, m_b, chunk_v,
                   slots, osem):
    wid = lax.axis_index("s") * _NC + lax.axis_index("c")
    B = idx_hbm.shape[0]
    D = table_t_hbm.shape[0]
    R = table_t_hbm.shape[1]
    n_chunks = (R + _CL - 1) // _CL
    last_chunk = n_chunks - 1
    r_phys = ((R + 127) // 128) * 128
    lanes16 = jnp.arange(_L, dtype=jnp.int32)

    pltpu.sync_copy(idx_hbm, all_idx)

    def scan_all(g, cnt):
        vec = all_idx[pl.ds(g * _L, _L)]
        mask = ((vec >> 10) & (_NW - 1)) == wid
        pos = lanes16 + g * _L
        keys = 1 - mask.astype(jnp.int32)
        _, ps = plsc.sort_key_val(keys, pos)
        m_b[pl.ds(cnt, _L)] = ps
        npick = jnp.sum(mask.astype(jnp.int32))
        return cnt + npick

    cnt = lax.fori_loop(0, B // _L, scan_all, jnp.int32(0))

    def per_chunk(t, carry):
        ch = wid + t * _NW

        @pl.when(ch < n_chunks)
        def _():
            lane0 = jnp.where(ch == last_chunk, r_phys - _CL, ch * _CL)
            lane0 = pl.multiple_of(lane0, 128)
            pltpu.sync_copy(table_t_hbm.at[:, pl.ds(lane0, _CL)], chunk_v)

            def per_group(s, e2):
                mbv = m_b[pl.ds(s * _L, _L)]
                mrv = plsc.load_gather(all_idx, [mbv])
                valid = (lanes16 + s * _L) < cnt
                m = jnp.logical_and((mrv >> 10) == ch, valid)
                mi = m.astype(jnp.int32)

                ej = e2
                for j in range(_L):
                    b = mbv[j]
                    l = mrv[j] - lane0
                    sbase = (ej & 3) * _OSTRIDE

                    @pl.when(mi[j] == 1)
                    def _(ej=ej, b=b, l=l, sbase=sbase):
                        @pl.when(ej >= 4)
                        def _():
                            pltpu.make_async_copy(
                                slots.at[pl.ds(0, _OSTRIDE)],
                                out_hbm.at[pl.ds(0, _OSTRIDE)],
                                osem,
                            ).wait()

                        for q in range(D // _L):
                            v = plsc.load_gather(
                                chunk_v,
                                [lanes16 + q * _L,
                                 jnp.full((_L,), l, jnp.int32)],
                            )
                            slots[pl.ds(sbase + q * _L, _L)] = v
                        pltpu.async_copy(
                            slots.at[pl.ds(sbase, _OSTRIDE)],
                            out_hbm.at[pl.ds(b * _OSTRIDE, _OSTRIDE)],
                            osem,
                        )

                    ej = ej + mi[j]
                return ej

            e = lax.fori_loop(0, (cnt + _L - 1) // _L, per_group,
                              jnp.int32(0))
            for k in range(4):
                @pl.when(e >= k + 1)
                def _():
                    pltpu.make_async_copy(
                        slots.at[pl.ds(0, _OSTRIDE)],
                        out_hbm.at[pl.ds(0, _OSTRIDE)],
                        osem,
                    ).wait()

        return carry

    lax.fori_loop(0, (n_chunks + _NW - 1) // _NW, per_chunk, jnp.int32(0))


def kernel(labels, train, embedding_table):
    B = labels.shape[0]
    D = embedding_table.shape[1]
    key = jax.random.key(42)
    drop_ids = jax.random.uniform(key, (B,)) < DROPOUT_PROB
    dropped = jnp.where(drop_ids, NUM_CLASSES, labels)
    idx = jnp.where(train != 0, dropped, labels).astype(jnp.int32)

    table_t = embedding_table.T
    cap = B + _L

    mesh = plsc.VectorSubcoreMesh(core_axis_name="c", subcore_axis_name="s")
    run = functools.partial(
        pl.kernel,
        _gather_kernel,
        mesh=mesh,
        compiler_params=pltpu.CompilerParams(needs_layout_passes=False),
        out_type=jax.ShapeDtypeStruct((B * _OSTRIDE,), jnp.float32),
        scratch_types=[
            pltpu.VMEM((B,), jnp.int32),
            pltpu.VMEM((cap,), jnp.int32),
            pltpu.VMEM((D, _CL), jnp.float32),
            pltpu.VMEM((4 * _OSTRIDE,), jnp.float32),
            pltpu.SemaphoreType.DMA,
        ],
    )()
    out_flat = run(idx, table_t)
    return out_flat.reshape(B, _OSTRIDE)[:, :D]


# speedup vs baseline: 1.4608x; 1.1442x over previous
import functools

import jax
import jax.numpy as jnp
from jax import lax
from jax.experimental import pallas as pl
from jax.experimental.pallas import tpu as pltpu
from jax.experimental.pallas import tpu_sc as plsc

NUM_CLASSES = 1000000
DROPOUT_PROB = 0.1

_info = plsc.get_sparse_core_info()
_NC, _NS = _info.num_cores, _info.num_subcores
_NW = _NC * _NS

_L = 16
_CL = 512
_CSH = 9
_OSTRIDE = 128


def _gather_kernel(idx_hbm, table_t_hbm, out_hbm, all_idx, m_b, chunk0,
                   chunk1, slots, csem0, csem1, osem):
    wid = lax.axis_index("s") * _NC + lax.axis_index("c")
    B = idx_hbm.shape[0]
    D = table_t_hbm.shape[0]
    R = table_t_hbm.shape[1]
    n_chunks = (R + _CL - 1) // _CL
    r_phys = ((R + 127) // 128) * 128

    pltpu.sync_copy(idx_hbm, all_idx)

    def scan_all(g, cnt):
        vec = all_idx[pl.ds(g * _L, _L)]
        mask = ((vec >> _CSH) & (_NW - 1)) == wid
        mi = mask.astype(jnp.int32)
        npick = jnp.sum(mi)

        @pl.when(npick > 0)
        def _():
            pos = jnp.arange(_L, dtype=jnp.int32) + g * _L
            _, ps = plsc.sort_key_val(1 - mi, pos)
            m_b[pl.ds(cnt, _L)] = ps

        return cnt + npick

    cnt = lax.fori_loop(0, B // _L, scan_all, jnp.int32(0))
    n_groups = (cnt + _L - 1) // _L

    def lane0_of(ch):
        return pl.multiple_of(jnp.minimum(ch * _CL, r_phys - _CL), 128)

    def process(ch, buf, sem):
        @pl.when(ch < n_chunks)
        def _():
            lane0 = lane0_of(ch)
            pltpu.sync_copy(table_t_hbm.at[:, pl.ds(lane0, _CL)], buf)

            def per_group(s, e2):
                mbv = m_b[pl.ds(s * _L, _L)]
                mrv = plsc.load_gather(all_idx, [mbv])
                lanes = jnp.arange(_L, dtype=jnp.int32)
                valid = (lanes + s * _L) < cnt
                mi = jnp.logical_and((mrv >> _CSH) == ch, valid).astype(
                    jnp.int32
                )
                nm = jnp.sum(mi)

                @pl.when(nm > 0)
                def _():
                    ej = e2
                    for j in range(_L):
                        b = mbv[j]
                        l = mrv[j] - lane0

                        @pl.when(mi[j] == 1)
                        def _(ej=ej, b=b, l=l):
                            sbase = (ej & 3) * _OSTRIDE

                            @pl.when(ej >= 4)
                            def _():
                                pltpu.make_async_copy(
                                    slots.at[pl.ds(0, _OSTRIDE)],
                                    out_hbm.at[pl.ds(0, _OSTRIDE)],
                                    osem,
                                ).wait()

                            for q in range(D // _L):
                                v = plsc.load_gather(
                                    buf,
                                    [jnp.arange(_L, dtype=jnp.int32)
                                     + q * _L,
                                     jnp.full((_L,), l, jnp.int32)],
                                )
                                slots[pl.ds(sbase + q * _L, _L)] = v
                            pltpu.async_copy(
                                slots.at[pl.ds(sbase, _OSTRIDE)],
                                out_hbm.at[pl.ds(b * _OSTRIDE, _OSTRIDE)],
                                osem,
                            )

                        ej = ej + mi[j]
                return e2 + nm

            e = lax.fori_loop(0, n_groups, per_group, jnp.int32(0))
            for k in range(4):
                @pl.when(e >= k + 1)
                def _():
                    pltpu.make_async_copy(
                        slots.at[pl.ds(0, _OSTRIDE)],
                        out_hbm.at[pl.ds(0, _OSTRIDE)],
                        osem,
                    ).wait()

    n_rounds = (n_chunks + _NW - 1) // _NW

    def per_round(t, carry):
        process(wid + t * _NW, chunk0, csem0)
        return carry

    lax.fori_loop(0, n_rounds, per_round, jnp.int32(0))


def kernel(labels, train, embedding_table):
    B = labels.shape[0]
    D = embedding_table.shape[1]
    key = jax.random.key(42)
    drop_ids = jax.random.uniform(key, (B,)) < DROPOUT_PROB
    dropped = jnp.where(drop_ids, NUM_CLASSES, labels)
    idx = jnp.where(train != 0, dropped, labels).astype(jnp.int32)

    table_t = embedding_table.T
    cap = B + _L

    mesh = plsc.VectorSubcoreMesh(core_axis_name="c", subcore_axis_name="s")
    run = functools.partial(
        pl.kernel,
        _gather_kernel,
        mesh=mesh,
        compiler_params=pltpu.CompilerParams(needs_layout_passes=False),
        out_type=jax.ShapeDtypeStruct((B * _OSTRIDE,), jnp.float32),
        scratch_types=[
            pltpu.VMEM((B,), jnp.int32),
            pltpu.VMEM((cap,), jnp.int32),
            pltpu.VMEM((D, _CL), jnp.float32),
            pltpu.VMEM((D, _CL), jnp.float32),
            pltpu.VMEM((4 * _OSTRIDE,), jnp.float32),
            pltpu.SemaphoreType.DMA,
            pltpu.SemaphoreType.DMA,
            pltpu.SemaphoreType.DMA,
        ],
    )()
    out_flat = run(idx, table_t)
    return out_flat.reshape(B, _OSTRIDE)[:, :D]


# speedup vs baseline: 2.0229x; 1.3848x over previous
import functools

import jax
import jax.numpy as jnp
from jax import lax
from jax.experimental import pallas as pl
from jax.experimental.pallas import tpu as pltpu
from jax.experimental.pallas import tpu_sc as plsc

NUM_CLASSES = 1000000
DROPOUT_PROB = 0.1

_info = plsc.get_sparse_core_info()
_NC, _NS = _info.num_cores, _info.num_subcores
_NW = _NC * _NS

_L = 16
_CL = 512
_CSH = 9
_OSTRIDE = 128


def _gather_kernel(idx_hbm, table_t_hbm, out_hbm, all_idx, m_b, chunk0,
                   chunk1, slots, csem0, csem1, osem):
    wid = lax.axis_index("s") * _NC + lax.axis_index("c")
    B = idx_hbm.shape[0]
    D = table_t_hbm.shape[0]
    R = table_t_hbm.shape[1]
    n_chunks = (R + _CL - 1) // _CL
    r_phys = ((R + 127) // 128) * 128

    pltpu.sync_copy(idx_hbm, all_idx)

    def scan_all(g, cnt):
        vec = all_idx[pl.ds(g * _L, _L)]
        mask = ((vec >> _CSH) & (_NW - 1)) == wid
        mi = mask.astype(jnp.int32)
        npick = jnp.sum(mi)

        @pl.when(npick > 0)
        def _():
            pos = jnp.arange(_L, dtype=jnp.int32) + g * _L
            _, ps = plsc.sort_key_val(1 - mi, pos)
            m_b[pl.ds(cnt, _L)] = ps

        return cnt + npick

    cnt = lax.fori_loop(0, B // _L, scan_all, jnp.int32(0))
    n_groups = (cnt + _L - 1) // _L

    def lane0_of(ch):
        return pl.multiple_of(jnp.minimum(ch * _CL, r_phys - _CL), 128)

    def issue(ch, buf, sem):
        @pl.when(ch < n_chunks)
        def _():
            pltpu.async_copy(
                table_t_hbm.at[:, pl.ds(lane0_of(ch), _CL)], buf, sem
            )

    def process(ch, buf, sem):
        @pl.when(ch < n_chunks)
        def _():
            lane0 = lane0_of(ch)
            pltpu.make_async_copy(
                table_t_hbm.at[:, pl.ds(lane0, _CL)], buf, sem
            ).wait()

            def per_group(s, e2):
                mbv = m_b[pl.ds(s * _L, _L)]
                mrv = plsc.load_gather(all_idx, [mbv])
                lanes = jnp.arange(_L, dtype=jnp.int32)
                valid = (lanes + s * _L) < cnt
                mi = jnp.logical_and((mrv >> _CSH) == ch, valid).astype(
                    jnp.int32
                )
                nm = jnp.sum(mi)

                @pl.when(nm > 0)
                def _():
                    ej = e2
                    for j in range(_L):
                        b = mbv[j]
                        l = mrv[j] - lane0

                        @pl.when(mi[j] == 1)
                        def _(ej=ej, b=b, l=l):
                            sbase = (ej & 3) * _OSTRIDE

                            @pl.when(ej >= 4)
                            def _():
                                pltpu.make_async_copy(
                                    slots.at[pl.ds(0, _OSTRIDE)],
                                    out_hbm.at[pl.ds(0, _OSTRIDE)],
                                    osem,
                                ).wait()

                            for q in range(D // _L):
                                v = plsc.load_gather(
                                    buf,
                                    [jnp.arange(_L, dtype=jnp.int32)
                                     + q * _L,
                                     jnp.full((_L,), l, jnp.int32)],
                                )
                                slots[pl.ds(sbase + q * _L, _L)] = v
                            pltpu.async_copy(
                                slots.at[pl.ds(sbase, _OSTRIDE)],
                                out_hbm.at[pl.ds(b * _OSTRIDE, _OSTRIDE)],
                                osem,
                            )

                        ej = ej + mi[j]
                return e2 + nm

            e = lax.fori_loop(0, n_groups, per_group, jnp.int32(0))
            for k in range(4):
                @pl.when(e >= k + 1)
                def _():
                    pltpu.make_async_copy(
                        slots.at[pl.ds(0, _OSTRIDE)],
                        out_hbm.at[pl.ds(0, _OSTRIDE)],
                        osem,
                    ).wait()

    n_rounds = (n_chunks + _NW - 1) // _NW
    n_half = (n_rounds + 1) // 2

    issue(wid, chunk0, csem0)

    def per_pair(u, carry):
        t0 = 2 * u
        ch0 = wid + t0 * _NW
        ch1 = wid + (t0 + 1) * _NW
        ch2 = wid + (t0 + 2) * _NW
        issue(ch1, chunk1, csem1)
        process(ch0, chunk0, csem0)
        issue(ch2, chunk0, csem0)
        process(ch1, chunk1, csem1)
        return carry

    lax.fori_loop(0, n_half, per_pair, jnp.int32(0))


def kernel(labels, train, embedding_table):
    B = labels.shape[0]
    D = embedding_table.shape[1]
    key = jax.random.key(42)
    drop_ids = jax.random.uniform(key, (B,)) < DROPOUT_PROB
    dropped = jnp.where(drop_ids, NUM_CLASSES, labels)
    idx = jnp.where(train != 0, dropped, labels).astype(jnp.int32)

    table_t = embedding_table.T
    cap = B + _L

    mesh = plsc.VectorSubcoreMesh(core_axis_name="c", subcore_axis_name="s")
    run = functools.partial(
        pl.kernel,
        _gather_kernel,
        mesh=mesh,
        compiler_params=pltpu.CompilerParams(needs_layout_passes=False),
        out_type=jax.ShapeDtypeStruct((B * _OSTRIDE,), jnp.float32),
        scratch_types=[
            pltpu.VMEM((B,), jnp.int32),
            pltpu.VMEM((cap,), jnp.int32),
            pltpu.VMEM((D, _CL), jnp.float32),
            pltpu.VMEM((D, _CL), jnp.float32),
            pltpu.VMEM((4 * _OSTRIDE,), jnp.float32),
            pltpu.SemaphoreType.DMA,
            pltpu.SemaphoreType.DMA,
            pltpu.SemaphoreType.DMA,
        ],
    )()
    out_flat = run(idx, table_t)
    return out_flat.reshape(B, _OSTRIDE)[:, :D]


# speedup vs baseline: 2.0258x; 1.0014x over previous
import functools

import jax
import jax.numpy as jnp
from jax import lax
from jax.experimental import pallas as pl
from jax.experimental.pallas import tpu as pltpu
from jax.experimental.pallas import tpu_sc as plsc

NUM_CLASSES = 1000000
DROPOUT_PROB = 0.1

_info = plsc.get_sparse_core_info()
_NC, _NS = _info.num_cores, _info.num_subcores
_NW = _NC * _NS

_L = 16
_CL = 512
_CSH = 9
_OSTRIDE = 128


def _gather_kernel(idx_hbm, table_t_hbm, out_hbm, all_idx, m_b, chunk0,
                   chunk1, slots, csem0, csem1, osem):
    wid = lax.axis_index("s") * _NC + lax.axis_index("c")
    B = idx_hbm.shape[0]
    D = table_t_hbm.shape[0]
    R = table_t_hbm.shape[1]
    n_chunks = (R + _CL - 1) // _CL
    r_phys = ((R + 127) // 128) * 128

    def lane0_of(ch):
        return pl.multiple_of(jnp.minimum(ch * _CL, r_phys - _CL), 128)

    def issue(ch, buf, sem):
        @pl.when(ch < n_chunks)
        def _():
            pltpu.async_copy(
                table_t_hbm.at[:, pl.ds(lane0_of(ch), _CL)], buf, sem
            )

    pltpu.sync_copy(idx_hbm, all_idx)

    def scan_all(g, cnt):
        vec = all_idx[pl.ds(g * _L, _L)]
        mask = ((vec >> _CSH) & (_NW - 1)) == wid
        mi = mask.astype(jnp.int32)
        npick = jnp.sum(mi)

        @pl.when(npick > 0)
        def _():
            pos = jnp.arange(_L, dtype=jnp.int32) + g * _L
            _, ps = plsc.sort_key_val(1 - mi, pos)
            m_b[pl.ds(cnt, _L)] = ps

        return cnt + npick

    cnt = lax.fori_loop(0, B // _L, scan_all, jnp.int32(0))
    n_groups = (cnt + _L - 1) // _L

    def process(ch, buf, sem):
        @pl.when(ch < n_chunks)
        def _():
            lane0 = lane0_of(ch)
            pltpu.make_async_copy(
                table_t_hbm.at[:, pl.ds(lane0, _CL)], buf, sem
            ).wait()

            def per_group(s, e2):
                mbv = m_b[pl.ds(s * _L, _L)]
                mrv = plsc.load_gather(all_idx, [mbv])
                lanes = jnp.arange(_L, dtype=jnp.int32)
                valid = (lanes + s * _L) < cnt
                mi = jnp.logical_and((mrv >> _CSH) == ch, valid).astype(
                    jnp.int32
                )
                nm = jnp.sum(mi)

                @pl.when(nm > 0)
                def _():
                    ej = e2
                    for j in range(_L):
                        b = mbv[j]
                        l = mrv[j] - lane0

                        @pl.when(mi[j] == 1)
                        def _(ej=ej, b=b, l=l):
                            sbase = (ej & 3) * _OSTRIDE

                            @pl.when(ej >= 4)
                            def _():
                                pltpu.make_async_copy(
                                    slots.at[pl.ds(0, _OSTRIDE)],
                                    out_hbm.at[pl.ds(0, _OSTRIDE)],
                                    osem,
                                ).wait()

                            for q in range(D // _L):
                                v = plsc.load_gather(
                                    buf,
                                    [jnp.arange(_L, dtype=jnp.int32)
                                     + q * _L,
                                     jnp.full((_L,), l, jnp.int32)],
                                )
                                slots[pl.ds(sbase + q * _L, _L)] = v
                            pltpu.async_copy(
                                slots.at[pl.ds(sbase, _OSTRIDE)],
                                out_hbm.at[pl.ds(b * _OSTRIDE, _OSTRIDE)],
                                osem,
                            )

                        ej = ej + mi[j]
                return e2 + nm

            e = lax.fori_loop(0, n_groups, per_group, jnp.int32(0))
            for k in range(4):
                @pl.when(e >= k + 1)
                def _():
                    pltpu.make_async_copy(
                        slots.at[pl.ds(0, _OSTRIDE)],
                        out_hbm.at[pl.ds(0, _OSTRIDE)],
                        osem,
                    ).wait()

    n_rounds = (n_chunks + _NW - 1) // _NW
    n_half = (n_rounds + 1) // 2

    issue(wid, chunk0, csem0)

    def per_pair(u, carry):
        t0 = 2 * u
        ch0 = wid + t0 * _NW
        ch1 = wid + (t0 + 1) * _NW
        ch2 = wid + (t0 + 2) * _NW
        issue(ch1, chunk1, csem1)
        process(ch0, chunk0, csem0)
        issue(ch2, chunk0, csem0)
        process(ch1, chunk1, csem1)
        return carry

    lax.fori_loop(0, n_half, per_pair, jnp.int32(0))


def kernel(labels, train, embedding_table):
    B = labels.shape[0]
    D = embedding_table.shape[1]
    key = jax.random.key(42)
    drop_ids = jax.random.uniform(key, (B,)) < DROPOUT_PROB
    dropped = jnp.where(drop_ids, NUM_CLASSES, labels)
    idx = jnp.where(train != 0, dropped, labels).astype(jnp.int32)

    table_t = embedding_table.T
    cap = B + _L

    mesh = plsc.VectorSubcoreMesh(core_axis_name="c", subcore_axis_name="s")
    run = functools.partial(
        pl.kernel,
        _gather_kernel,
        mesh=mesh,
        compiler_params=pltpu.CompilerParams(needs_layout_passes=False),
        out_type=jax.ShapeDtypeStruct((B * _OSTRIDE,), jnp.float32),
        scratch_types=[
            pltpu.VMEM((B,), jnp.int32),
            pltpu.VMEM((cap,), jnp.int32),
            pltpu.VMEM((D, _CL), jnp.float32),
            pltpu.VMEM((D, _CL), jnp.float32),
            pltpu.VMEM((4 * _OSTRIDE,), jnp.float32),
            pltpu.SemaphoreType.DMA,
            pltpu.SemaphoreType.DMA,
            pltpu.SemaphoreType.DMA,
        ],
    )()
    out_flat = run(idx, table_t)
    return out_flat.reshape(B, _OSTRIDE)[:, :D]
